# baseline (device time: 374320 ns/iter reference)
import jax
import jax.numpy as jnp
from jax import lax
from jax.experimental import pallas as pl
from jax.experimental.pallas import tpu as pltpu

N_DEV = 4
M_BLK = 1024
M_ALL = N_DEV * M_BLK
K_PER = 1024
N_TOT = 8192
W = 1024
S = N_TOT // W
H = W // 2
Q = W // 4


def _rs_body(x_hbm, w_hbm, out_ref, amax_ref,
             x_stage, x_vmem, w_stage, w_vmem, p_buf,
             cwa_ref, cwb_ref, ccwa_ref, ccwb_ref,
             x_sems, w_sems,
             cwa_send, cwa_recv, cwb_send, cwb_recv,
             ccwa_send, ccwa_recv, ccwb_send, ccwb_recv):
    my = lax.axis_index("i")
    left = lax.rem(my + N_DEV - 1, N_DEV)
    right = lax.rem(my + 1, N_DEV)
    j = pl.program_id(0)
    cur = lax.rem(j, 2)
    nxt = lax.rem(j + 1, 2)

    barrier_sem = pltpu.get_barrier_semaphore()
    for nbr in (left, right):
        pl.semaphore_signal(
            barrier_sem, inc=1,
            device_id=(nbr,), device_id_type=pl.DeviceIdType.MESH,
        )
    pl.semaphore_wait(barrier_sem, 2)

    def gemm_into(slot, w_slot):
        w_vmem[w_slot] = w_stage[w_slot].astype(jnp.bfloat16)
        for c in range(N_DEV):
            p_buf[pl.ds(slot * M_ALL + c * M_BLK, M_BLK), :] = jnp.dot(
                x_vmem[pl.ds(c * M_BLK, M_BLK), :],
                w_vmem[w_slot],
                preferred_element_type=jnp.float32,
            ).astype(jnp.bfloat16)

    def x_chunk_copy(c):
        return pltpu.make_async_copy(
            x_hbm.at[pl.ds(c * M_BLK, M_BLK), :],
            x_stage.at[c % 2],
            x_sems.at[c % 2],
        )

    @pl.when(j == 0)
    def _():
        wc = pltpu.make_async_copy(
            w_hbm.at[:, pl.ds(0, W)], w_stage.at[0], w_sems.at[0]
        )
        wc.start()
        x_chunk_copy(0).start()
        x_chunk_copy(1).start()
        for c in range(N_DEV):
            x_chunk_copy(c).wait()
            x_vmem[pl.ds(c * M_BLK, M_BLK), :] = x_stage[c % 2].astype(
                jnp.bfloat16
            )
            if c + 2 < N_DEV:
                x_chunk_copy(c + 2).start()
        wc.wait()
        gemm_into(0, 0)

    @pl.when(j + 1 < S)
    def _():
        pltpu.make_async_copy(
            w_hbm.at[:, pl.ds((j + 1) * W, W)], w_stage.at[nxt],
            w_sems.at[nxt],
        ).start()

    flows = [
        dict(comm=cwa_ref, ss=cwa_send, rs=cwa_recv, dev=right, lo=0,
             sign=-1),
        dict(comm=ccwa_ref, ss=ccwa_send, rs=ccwa_recv, dev=left, lo=2 * Q,
             sign=1),
        dict(comm=cwb_ref, ss=cwb_send, rs=cwb_recv, dev=right, lo=Q,
             sign=-1),
        dict(comm=ccwb_ref, ss=ccwb_send, rs=ccwb_recv, dev=left, lo=3 * Q,
             sign=1),
    ]

    def p_q(c, f):
        lo = f["lo"]
        return p_buf[pl.ds(cur * M_ALL + c * M_BLK, M_BLK), lo:lo + Q]

    def rdma(f, h):
        return pltpu.make_async_remote_copy(
            src_ref=f["comm"].at[h % 2],
            dst_ref=f["comm"].at[(h + 1) % 2],
            send_sem=f["ss"].at[h % 2],
            recv_sem=f["rs"].at[(h + 1) % 2],
            device_id=(f["dev"],),
            device_id_type=pl.DeviceIdType.MESH,
        )

    def chunk(f, h, recv):
        off = (1 + h + (1 if recv else 0)) * f["sign"]
        return lax.rem(my + off + 2 * N_DEV, N_DEV)

    for f in flows:
        f["comm"][0] = p_q(chunk(f, 0, recv=False), f)
        rdma(f, 0).start()

    @pl.when(j + 1 < S)
    def _():
        pltpu.make_async_copy(
            w_hbm.at[:, pl.ds((j + 1) * W, W)], w_stage.at[nxt],
            w_sems.at[nxt],
        ).wait()

        @pl.when(nxt == 1)
        def _():
            gemm_into(1, 1)

        @pl.when(nxt == 0)
        def _():
            gemm_into(0, 0)

    local = None
    for h in range(N_DEV - 1):
        recv_slot = (h + 1) % 2
        for f in flows:
            rdma(f, h).wait()
            c = chunk(f, h, recv=True)
            if h < N_DEV - 2:
                f["comm"][recv_slot] = f["comm"][recv_slot] + p_q(c, f)
                rdma(f, h + 1).start()
            else:
                acc = f["comm"][recv_slot].astype(jnp.float32) + p_q(
                    c, f
                ).astype(jnp.float32)
                lo = f["lo"]
                acc_bf = acc.astype(jnp.bfloat16)
                out_ref[:, lo:lo + Q] = acc_bf
                m = jnp.max(acc_bf.astype(jnp.float32))
                local = m if local is None else jnp.maximum(local, m)

    local_max = jnp.full((8, 128), jnp.maximum(local, 0.0))
    prev = jnp.where(j == 0, jnp.zeros((8, 128), jnp.float32),
                     amax_ref[...])
    amax_ref[...] = jnp.maximum(prev, local_max)


def _rs_call(x, w_mat):
    return pl.pallas_call(
        _rs_body,
        grid=(S,),
        in_specs=[
            pl.BlockSpec(memory_space=pltpu.MemorySpace.HBM),
            pl.BlockSpec(memory_space=pltpu.MemorySpace.HBM),
        ],
        out_specs=[
            pl.BlockSpec((M_BLK, W), lambda j: (0, j)),
            pl.BlockSpec((8, 128), lambda j: (0, 0)),
        ],
        out_shape=[
            jax.ShapeDtypeStruct((M_BLK, N_TOT), jnp.bfloat16),
            jax.ShapeDtypeStruct((8, 128), jnp.float32),
        ],
        scratch_shapes=[
            pltpu.VMEM((2, M_BLK, K_PER), jnp.float32),
            pltpu.VMEM((M_ALL, K_PER), jnp.bfloat16),
            pltpu.VMEM((2, K_PER, W), jnp.float32),
            pltpu.VMEM((2, K_PER, W), jnp.bfloat16),
            pltpu.VMEM((2 * M_ALL, W), jnp.bfloat16),
            pltpu.VMEM((2, M_BLK, Q), jnp.bfloat16),
            pltpu.VMEM((2, M_BLK, Q), jnp.bfloat16),
            pltpu.VMEM((2, M_BLK, Q), jnp.bfloat16),
            pltpu.VMEM((2, M_BLK, Q), jnp.bfloat16),
            pltpu.SemaphoreType.DMA((2,)),
            pltpu.SemaphoreType.DMA((2,)),
            pltpu.SemaphoreType.DMA((2,)),
            pltpu.SemaphoreType.DMA((2,)),
            pltpu.SemaphoreType.DMA((2,)),
            pltpu.SemaphoreType.DMA((2,)),
            pltpu.SemaphoreType.DMA((2,)),
            pltpu.SemaphoreType.DMA((2,)),
            pltpu.SemaphoreType.DMA((2,)),
            pltpu.SemaphoreType.DMA((2,)),
        ],
        compiler_params=pltpu.CompilerParams(
            collective_id=0,
            dimension_semantics=("arbitrary",),
            vmem_limit_bytes=100 * 1024 * 1024,
        ),
    )(x, w_mat)


WE = 2048
SE = N_TOT // WE


def _ep_body(amax_in, y_ref, out_ref, gmax_ref, comm_ref,
             send_sems, recv_sems):
    j = pl.program_id(0)
    my = lax.axis_index("i")

    @pl.when(j == 0)
    def _():
        barrier_sem = pltpu.get_barrier_semaphore()
        for k in range(1, N_DEV):
            pl.semaphore_signal(
                barrier_sem, inc=1,
                device_id=(lax.rem(my + k, N_DEV),),
                device_id_type=pl.DeviceIdType.MESH,
            )
        pl.semaphore_wait(barrier_sem, N_DEV - 1)

        rdmas = []
        for k in range(1, N_DEV):
            target = lax.rem(my + k, N_DEV)
            rdma = pltpu.make_async_remote_copy(
                src_ref=amax_in,
                dst_ref=comm_ref.at[k],
                send_sem=send_sems.at[k],
                recv_sem=recv_sems.at[k],
                device_id=(target,),
                device_id_type=pl.DeviceIdType.MESH,
            )
            rdma.start()
            rdmas.append(rdma)
        for rdma in rdmas:
            rdma.wait()

        result = amax_in[...]
        for k in range(1, N_DEV):
            result = jnp.maximum(result, comm_ref[k])
        gmax_ref[...] = result

    amax = jnp.max(gmax_ref[...])
    y = jnp.maximum(y_ref[...].astype(jnp.float32), 0.0)
    q = (y * (448.0 / amax)).astype(jnp.float8_e4m3fn)
    out_ref[...] = q.astype(jnp.float32) * (amax / 448.0)


def _ep_call(amax_local, y):
    return pl.pallas_call(
        _ep_body,
        grid=(SE,),
        in_specs=[
            pl.BlockSpec((8, 128), lambda j: (0, 0)),
            pl.BlockSpec((M_BLK, WE), lambda j: (0, j)),
        ],
        out_specs=pl.BlockSpec((M_BLK, WE), lambda j: (0, j)),
        out_shape=jax.ShapeDtypeStruct((M_BLK, N_TOT), jnp.float32),
        scratch_shapes=[
            pltpu.VMEM((8, 128), jnp.float32),
            pltpu.VMEM((N_DEV, 8, 128), jnp.float32),
            pltpu.SemaphoreType.DMA((N_DEV,)),
            pltpu.SemaphoreType.DMA((N_DEV,)),
        ],
        compiler_params=pltpu.CompilerParams(
            collective_id=1,
            dimension_semantics=("arbitrary",),
        ),
    )(amax_local, y)


def kernel(x, w_mat):
    y, amax_local = _rs_call(x, w_mat)
    return _ep_call(amax_local, y)


# device time: 353014 ns/iter; 1.0604x vs baseline; 1.0604x over previous
import jax
import jax.numpy as jnp
from jax import lax
from jax.experimental import pallas as pl
from jax.experimental.pallas import tpu as pltpu

N_DEV = 4
M_BLK = 1024
M_ALL = N_DEV * M_BLK
K_PER = 1024
N_TOT = 8192
W = 1024
S = N_TOT // W
H = W // 2
Q = W // 4


def _rs_body(x_hbm, w_hbm, out_ref, amax_ref,
             x_stage, x_vmem, w_stage, w_vmem, p_buf,
             cwa_ref, cwb_ref, ccwa_ref, ccwb_ref,
             x_sems, w_sems,
             cwa_send, cwa_recv, cwb_send, cwb_recv,
             ccwa_send, ccwa_recv, ccwb_send, ccwb_recv):
    my = lax.axis_index("i")
    left = lax.rem(my + N_DEV - 1, N_DEV)
    right = lax.rem(my + 1, N_DEV)
    j = pl.program_id(0)
    cur = lax.rem(j, 2)
    nxt = lax.rem(j + 1, 2)

    barrier_sem = pltpu.get_barrier_semaphore()
    for nbr in (left, right):
        pl.semaphore_signal(
            barrier_sem, inc=1,
            device_id=(nbr,), device_id_type=pl.DeviceIdType.MESH,
        )
    pl.semaphore_wait(barrier_sem, 2)

    def gemm_into(slot, w_slot):
        w_vmem[w_slot] = w_stage[w_slot].astype(jnp.bfloat16)
        for c in range(N_DEV):
            p_buf[pl.ds(slot * M_ALL + c * M_BLK, M_BLK), :] = jnp.dot(
                x_vmem[pl.ds(c * M_BLK, M_BLK), :],
                w_vmem[w_slot],
                preferred_element_type=jnp.float32,
            ).astype(jnp.bfloat16)

    def x_chunk_copy(c):
        return pltpu.make_async_copy(
            x_hbm.at[pl.ds(c * M_BLK, M_BLK), :],
            x_stage.at[c % 2],
            x_sems.at[c % 2],
        )

    @pl.when(j == 0)
    def _():
        wc = pltpu.make_async_copy(
            w_hbm.at[:, pl.ds(0, W)], w_stage.at[0], w_sems.at[0]
        )
        wc.start()
        x_chunk_copy(0).start()
        x_chunk_copy(1).start()
        for c in range(N_DEV):
            x_chunk_copy(c).wait()
            x_vmem[pl.ds(c * M_BLK, M_BLK), :] = x_stage[c % 2].astype(
                jnp.bfloat16
            )
            if c + 2 < N_DEV:
                x_chunk_copy(c + 2).start()
        wc.wait()
        gemm_into(0, 0)

    @pl.when(j + 1 < S)
    def _():
        pltpu.make_async_copy(
            w_hbm.at[:, pl.ds((j + 1) * W, W)], w_stage.at[nxt],
            w_sems.at[nxt],
        ).start()

    flows = [
        dict(comm=cwa_ref, ss=cwa_send, rs=cwa_recv, dev=right, lo=0,
             sign=-1),
        dict(comm=ccwa_ref, ss=ccwa_send, rs=ccwa_recv, dev=left, lo=2 * Q,
             sign=1),
        dict(comm=cwb_ref, ss=cwb_send, rs=cwb_recv, dev=right, lo=Q,
             sign=-1),
        dict(comm=ccwb_ref, ss=ccwb_send, rs=ccwb_recv, dev=left, lo=3 * Q,
             sign=1),
    ]

    def p_q(c, f):
        lo = f["lo"]
        return p_buf[pl.ds(cur * M_ALL + c * M_BLK, M_BLK), lo:lo + Q]

    def rdma(f, h):
        return pltpu.make_async_remote_copy(
            src_ref=f["comm"].at[h % 2],
            dst_ref=f["comm"].at[(h + 1) % 2],
            send_sem=f["ss"].at[h % 2],
            recv_sem=f["rs"].at[(h + 1) % 2],
            device_id=(f["dev"],),
            device_id_type=pl.DeviceIdType.MESH,
        )

    def chunk(f, h, recv):
        off = (1 + h + (1 if recv else 0)) * f["sign"]
        return lax.rem(my + off + 2 * N_DEV, N_DEV)

    for f in flows:
        f["comm"][0] = p_q(chunk(f, 0, recv=False), f)
        rdma(f, 0).start()

    @pl.when(j + 1 < S)
    def _():
        pltpu.make_async_copy(
            w_hbm.at[:, pl.ds((j + 1) * W, W)], w_stage.at[nxt],
            w_sems.at[nxt],
        ).wait()

        @pl.when(nxt == 1)
        def _():
            gemm_into(1, 1)

        @pl.when(nxt == 0)
        def _():
            gemm_into(0, 0)

    local = None
    for h in range(N_DEV - 1):
        recv_slot = (h + 1) % 2
        for f in flows:
            rdma(f, h).wait()
            c = chunk(f, h, recv=True)
            if h < N_DEV - 2:
                f["comm"][recv_slot] = f["comm"][recv_slot] + p_q(c, f)
                rdma(f, h + 1).start()
            else:
                acc = f["comm"][recv_slot].astype(jnp.float32) + p_q(
                    c, f
                ).astype(jnp.float32)
                lo = f["lo"]
                acc_bf = acc.astype(jnp.bfloat16)
                out_ref[:, lo:lo + Q] = acc_bf
                m = jnp.max(acc_bf.astype(jnp.float32))
                local = m if local is None else jnp.maximum(local, m)

    local_max = jnp.full((8, 128), jnp.maximum(local, 0.0))
    prev = jnp.where(j == 0, jnp.zeros((8, 128), jnp.float32),
                     amax_ref[...])
    amax_ref[...] = jnp.maximum(prev, local_max)


def _rs_call(x, w_mat):
    return pl.pallas_call(
        _rs_body,
        grid=(S,),
        in_specs=[
            pl.BlockSpec(memory_space=pltpu.MemorySpace.HBM),
            pl.BlockSpec(memory_space=pltpu.MemorySpace.HBM),
        ],
        out_specs=[
            pl.BlockSpec((M_BLK, W), lambda j: (0, j)),
            pl.BlockSpec((8, 128), lambda j: (0, 0)),
        ],
        out_shape=[
            jax.ShapeDtypeStruct((M_BLK, N_TOT), jnp.bfloat16),
            jax.ShapeDtypeStruct((8, 128), jnp.float32),
        ],
        scratch_shapes=[
            pltpu.VMEM((2, M_BLK, K_PER), jnp.float32),
            pltpu.VMEM((M_ALL, K_PER), jnp.bfloat16),
            pltpu.VMEM((2, K_PER, W), jnp.float32),
            pltpu.VMEM((2, K_PER, W), jnp.bfloat16),
            pltpu.VMEM((2 * M_ALL, W), jnp.bfloat16),
            pltpu.VMEM((2, M_BLK, Q), jnp.bfloat16),
            pltpu.VMEM((2, M_BLK, Q), jnp.bfloat16),
            pltpu.VMEM((2, M_BLK, Q), jnp.bfloat16),
            pltpu.VMEM((2, M_BLK, Q), jnp.bfloat16),
            pltpu.SemaphoreType.DMA((2,)),
            pltpu.SemaphoreType.DMA((2,)),
            pltpu.SemaphoreType.DMA((2,)),
            pltpu.SemaphoreType.DMA((2,)),
            pltpu.SemaphoreType.DMA((2,)),
            pltpu.SemaphoreType.DMA((2,)),
            pltpu.SemaphoreType.DMA((2,)),
            pltpu.SemaphoreType.DMA((2,)),
            pltpu.SemaphoreType.DMA((2,)),
            pltpu.SemaphoreType.DMA((2,)),
        ],
        compiler_params=pltpu.CompilerParams(
            collective_id=0,
            dimension_semantics=("arbitrary",),
            vmem_limit_bytes=100 * 1024 * 1024,
        ),
    )(x, w_mat)


def _amax_body(in_ref, out_ref, comm_ref, send_sems, recv_sems):
    my = lax.axis_index("i")

    barrier_sem = pltpu.get_barrier_semaphore()
    for k in range(1, N_DEV):
        pl.semaphore_signal(
            barrier_sem, inc=1,
            device_id=(lax.rem(my + k, N_DEV),),
            device_id_type=pl.DeviceIdType.MESH,
        )
    pl.semaphore_wait(barrier_sem, N_DEV - 1)

    rdmas = []
    for k in range(1, N_DEV):
        target = lax.rem(my + k, N_DEV)
        rdma = pltpu.make_async_remote_copy(
            src_ref=in_ref,
            dst_ref=comm_ref.at[k],
            send_sem=send_sems.at[k],
            recv_sem=recv_sems.at[k],
            device_id=(target,),
            device_id_type=pl.DeviceIdType.MESH,
        )
        rdma.start()
        rdmas.append(rdma)
    for rdma in rdmas:
        rdma.wait()

    result = in_ref[...]
    for k in range(1, N_DEV):
        result = jnp.maximum(result, comm_ref[k])
    out_ref[...] = result


def _amax_call(amax_local):
    return pl.pallas_call(
        _amax_body,
        in_specs=[pl.BlockSpec(memory_space=pltpu.VMEM)],
        out_specs=pl.BlockSpec(memory_space=pltpu.VMEM),
        out_shape=jax.ShapeDtypeStruct((8, 128), jnp.float32),
        scratch_shapes=[
            pltpu.VMEM((N_DEV, 8, 128), jnp.float32),
            pltpu.SemaphoreType.DMA((N_DEV,)),
            pltpu.SemaphoreType.DMA((N_DEV,)),
        ],
        compiler_params=pltpu.CompilerParams(collective_id=1),
    )(amax_local)


def kernel(x, w_mat):
    y, amax_local = _rs_call(x, w_mat)
    amax = _amax_call(amax_local)[0, 0]

    y = jnp.maximum(y.astype(jnp.float32), 0.0)
    scale = amax / 448.0
    q = (y / scale).astype(jnp.float8_e4m3fn)
    q = lax.optimization_barrier(q)
    return q.astype(jnp.float32) * scale


# device time: 335438 ns/iter; 1.1159x vs baseline; 1.0524x over previous
import jax
import jax.numpy as jnp
from jax import lax
from jax.experimental import pallas as pl
from jax.experimental.pallas import tpu as pltpu

N_DEV = 4
M_BLK = 1024
M_ALL = N_DEV * M_BLK
K_PER = 1024
N_TOT = 8192
W = 1024
S = N_TOT // W
H = W // 2
Q = W // 4


def _rs_body(x_hbm, w_hbm, out_ref, amax_ref,
             x_stage, x_vmem, w_stage, w_vmem, p_buf,
             cwa_ref, cwb_ref, ccwa_ref, ccwb_ref,
             x_sems, w_sems,
             cwa_send, cwa_recv, cwb_send, cwb_recv,
             ccwa_send, ccwa_recv, ccwb_send, ccwb_recv):
    my = lax.axis_index("i")
    left = lax.rem(my + N_DEV - 1, N_DEV)
    right = lax.rem(my + 1, N_DEV)
    j = pl.program_id(0)
    cur = lax.rem(j, 2)
    nxt = lax.rem(j + 1, 2)

    barrier_sem = pltpu.get_barrier_semaphore()
    for nbr in (left, right):
        pl.semaphore_signal(
            barrier_sem, inc=1,
            device_id=(nbr,), device_id_type=pl.DeviceIdType.MESH,
        )
    pl.semaphore_wait(barrier_sem, 2)

    def gemm_into(slot, w_slot):
        w_vmem[w_slot] = w_stage[w_slot].astype(jnp.bfloat16)
        for c in range(N_DEV):
            p_buf[pl.ds(slot * M_ALL + c * M_BLK, M_BLK), :] = jnp.dot(
                x_vmem[pl.ds(c * M_BLK, M_BLK), :],
                w_vmem[w_slot],
                preferred_element_type=jnp.float32,
            ).astype(jnp.bfloat16)

    def x_chunk_copy(c):
        return pltpu.make_async_copy(
            x_hbm.at[pl.ds(c * M_BLK, M_BLK), :],
            x_stage.at[c % 2],
            x_sems.at[c % 2],
        )

    @pl.when(j == 0)
    def _():
        wc = pltpu.make_async_copy(
            w_hbm.at[:, pl.ds(0, W)], w_stage.at[0], w_sems.at[0]
        )
        wc.start()
        x_chunk_copy(0).start()
        x_chunk_copy(1).start()
        for c in range(N_DEV):
            x_chunk_copy(c).wait()
            x_vmem[pl.ds(c * M_BLK, M_BLK), :] = x_stage[c % 2].astype(
                jnp.bfloat16
            )
            if c + 2 < N_DEV:
                x_chunk_copy(c + 2).start()
        wc.wait()
        gemm_into(0, 0)

    @pl.when(j + 1 < S)
    def _():
        pltpu.make_async_copy(
            w_hbm.at[:, pl.ds((j + 1) * W, W)], w_stage.at[nxt],
            w_sems.at[nxt],
        ).start()

    flows = [
        dict(comm=cwa_ref, ss=cwa_send, rs=cwa_recv, dev=right, lo=0,
             sign=-1),
        dict(comm=ccwa_ref, ss=ccwa_send, rs=ccwa_recv, dev=left, lo=2 * Q,
             sign=1),
        dict(comm=cwb_ref, ss=cwb_send, rs=cwb_recv, dev=right, lo=Q,
             sign=-1),
        dict(comm=ccwb_ref, ss=ccwb_send, rs=ccwb_recv, dev=left, lo=3 * Q,
             sign=1),
    ]

    def p_q(c, f, slot):
        lo = f["lo"]
        return p_buf[pl.ds(slot * M_ALL + c * M_BLK, M_BLK), lo:lo + Q]

    def rdma(f, h):
        return pltpu.make_async_remote_copy(
            src_ref=f["comm"].at[h],
            dst_ref=f["comm"].at[h + 1],
            send_sem=f["ss"].at[h],
            recv_sem=f["rs"].at[h],
            device_id=(f["dev"],),
            device_id_type=pl.DeviceIdType.MESH,
        )

    def chunk(f, h, recv):
        off = (1 + h + (1 if recv else 0)) * f["sign"]
        return lax.rem(my + off + 2 * N_DEV, N_DEV)

    @pl.when(j >= 1)
    def _():
        for f in flows:
            rdma(f, 2).wait_send()

    @pl.when(j < S)
    def _():
        for f in flows:
            f["comm"][0] = p_q(chunk(f, 0, recv=False), f, cur)
            rdma(f, 0).start()

    @pl.when(j >= 1)
    def _():
        local = None
        for f in flows:
            rdma(f, 2).wait_recv()
            acc = f["comm"][3].astype(jnp.float32) + p_q(
                my, f, nxt
            ).astype(jnp.float32)
            lo = f["lo"]
            acc_bf = acc.astype(jnp.bfloat16)
            out_ref[:, lo:lo + Q] = acc_bf
            m = jnp.max(acc_bf.astype(jnp.float32))
            local = m if local is None else jnp.maximum(local, m)
        local_max = jnp.full((8, 128), jnp.maximum(local, 0.0))
        prev = jnp.where(j == 1, jnp.zeros((8, 128), jnp.float32),
                         amax_ref[...])
        amax_ref[...] = jnp.maximum(prev, local_max)

    @pl.when(j + 1 < S)
    def _():
        pltpu.make_async_copy(
            w_hbm.at[:, pl.ds((j + 1) * W, W)], w_stage.at[nxt],
            w_sems.at[nxt],
        ).wait()

        @pl.when(nxt == 1)
        def _():
            gemm_into(1, 1)

        @pl.when(nxt == 0)
        def _():
            gemm_into(0, 0)

    @pl.when(j < S)
    def _():
        for h in range(N_DEV - 2):
            for f in flows:
                rdma(f, h).wait()
                c = chunk(f, h, recv=True)
                f["comm"][h + 1] = f["comm"][h + 1] + p_q(c, f, cur)
                rdma(f, h + 1).start()


def _rs_call(x, w_mat):
    return pl.pallas_call(
        _rs_body,
        grid=(S + 1,),
        in_specs=[
            pl.BlockSpec(memory_space=pltpu.MemorySpace.HBM),
            pl.BlockSpec(memory_space=pltpu.MemorySpace.HBM),
        ],
        out_specs=[
            pl.BlockSpec((M_BLK, W), lambda j: (0, jnp.maximum(j - 1, 0))),
            pl.BlockSpec((8, 128), lambda j: (0, 0)),
        ],
        out_shape=[
            jax.ShapeDtypeStruct((M_BLK, N_TOT), jnp.bfloat16),
            jax.ShapeDtypeStruct((8, 128), jnp.float32),
        ],
        scratch_shapes=[
            pltpu.VMEM((2, M_BLK, K_PER), jnp.float32),
            pltpu.VMEM((M_ALL, K_PER), jnp.bfloat16),
            pltpu.VMEM((2, K_PER, W), jnp.float32),
            pltpu.VMEM((2, K_PER, W), jnp.bfloat16),
            pltpu.VMEM((2 * M_ALL, W), jnp.bfloat16),
            pltpu.VMEM((4, M_BLK, Q), jnp.bfloat16),
            pltpu.VMEM((4, M_BLK, Q), jnp.bfloat16),
            pltpu.VMEM((4, M_BLK, Q), jnp.bfloat16),
            pltpu.VMEM((4, M_BLK, Q), jnp.bfloat16),
            pltpu.SemaphoreType.DMA((2,)),
            pltpu.SemaphoreType.DMA((2,)),
            pltpu.SemaphoreType.DMA((3,)),
            pltpu.SemaphoreType.DMA((3,)),
            pltpu.SemaphoreType.DMA((3,)),
            pltpu.SemaphoreType.DMA((3,)),
            pltpu.SemaphoreType.DMA((3,)),
            pltpu.SemaphoreType.DMA((3,)),
            pltpu.SemaphoreType.DMA((3,)),
            pltpu.SemaphoreType.DMA((3,)),
        ],
        compiler_params=pltpu.CompilerParams(
            collective_id=0,
            dimension_semantics=("arbitrary",),
            vmem_limit_bytes=100 * 1024 * 1024,
        ),
    )(x, w_mat)


def _amax_body(in_ref, out_ref, comm_ref, send_sems, recv_sems):
    my = lax.axis_index("i")

    barrier_sem = pltpu.get_barrier_semaphore()
    for k in range(1, N_DEV):
        pl.semaphore_signal(
            barrier_sem, inc=1,
            device_id=(lax.rem(my + k, N_DEV),),
            device_id_type=pl.DeviceIdType.MESH,
        )
    pl.semaphore_wait(barrier_sem, N_DEV - 1)

    rdmas = []
    for k in range(1, N_DEV):
        target = lax.rem(my + k, N_DEV)
        rdma = pltpu.make_async_remote_copy(
            src_ref=in_ref,
            dst_ref=comm_ref.at[k],
            send_sem=send_sems.at[k],
            recv_sem=recv_sems.at[k],
            device_id=(target,),
            device_id_type=pl.DeviceIdType.MESH,
        )
        rdma.start()
        rdmas.append(rdma)
    for rdma in rdmas:
        rdma.wait()

    result = in_ref[...]
    for k in range(1, N_DEV):
        result = jnp.maximum(result, comm_ref[k])
    out_ref[...] = result


def _amax_call(amax_local):
    return pl.pallas_call(
        _amax_body,
        in_specs=[pl.BlockSpec(memory_space=pltpu.VMEM)],
        out_specs=pl.BlockSpec(memory_space=pltpu.VMEM),
        out_shape=jax.ShapeDtypeStruct((8, 128), jnp.float32),
        scratch_shapes=[
            pltpu.VMEM((N_DEV, 8, 128), jnp.float32),
            pltpu.SemaphoreType.DMA((N_DEV,)),
            pltpu.SemaphoreType.DMA((N_DEV,)),
        ],
        compiler_params=pltpu.CompilerParams(collective_id=1),
    )(amax_local)


def kernel(x, w_mat):
    y, amax_local = _rs_call(x, w_mat)
    amax = _amax_call(amax_local)[0, 0]

    y = jnp.maximum(y.astype(jnp.float32), 0.0)
    scale = amax / 448.0
    q = (y / scale).astype(jnp.float8_e4m3fn)
    q = lax.optimization_barrier(q)
    return q.astype(jnp.float32) * scale
